# baseline (speedup 1.0000x reference)
import functools

import jax
import jax.numpy as jnp
from jax import lax
from jax.experimental import pallas as pl
from jax.experimental.pallas import tpu as pltpu
from jax.experimental.pallas import tpu_sc as plsc
from jax.experimental.layout import Format, Layout, with_layout_constraint

NC = 2
NS = 16
NW = NC * NS
LANES = 16


def _make_kernel(B, L, V, D):
    rows_total = B * L
    per_w = rows_total // NW
    CH = 128
    n_chunks = per_w // CH
    chunks_per_seq = L // CH
    vecs_per_row = D // LANES

    mesh = plsc.VectorSubcoreMesh(core_axis_name="c", subcore_axis_name="s")

    @functools.partial(
        pl.kernel,
        mesh=mesh,
        out_type=jax.ShapeDtypeStruct((rows_total, D), jnp.float32),
        scratch_types=[
            pltpu.VMEM((2, CH), jnp.int32),
            pltpu.VMEM((2, CH, D), jnp.float32),
            pltpu.VMEM((2, CH, D), jnp.float32),
            pltpu.SemaphoreType.DMA,
            pltpu.SemaphoreType.DMA,
        ],
    )
    def k(idx_hbm, wt_hbm, pt_hbm, out_hbm, idx_v, g_v, pos_v, gsem, psem):
        cid = lax.axis_index("c")
        sid = lax.axis_index("s")
        wid = sid * NC + cid
        jbase = wid * per_w

        def fire(c, buf):
            pltpu.sync_copy(idx_hbm.at[pl.ds(jbase + c * CH, CH)],
                            idx_v.at[buf])

            def issue(g, _):
                ivec = idx_v[buf, pl.ds(g * LANES, LANES)]
                for j in range(LANES):
                    pltpu.async_copy(
                        wt_hbm.at[ivec[j]], g_v.at[buf, g * LANES + j], gsem
                    )
                return 0

            lax.fori_loop(0, CH // LANES, issue, 0)
            l0 = lax.rem(c, chunks_per_seq) * CH
            pltpu.async_copy(pt_hbm.at[pl.ds(l0, CH)], pos_v.at[buf], psem)

        def process(c, buf):
            pltpu.make_async_copy(
                pt_hbm.at[pl.ds(0, CH)], g_v.at[buf], gsem).wait()
            pltpu.make_async_copy(
                pt_hbm.at[pl.ds(0, CH)], pos_v.at[buf], psem).wait()

            def add_row(r, _):
                for u in range(vecs_per_row):
                    sl = pl.ds(u * LANES, LANES)
                    g_v[buf, r, sl] = g_v[buf, r, sl] + pos_v[buf, r, sl]
                return 0

            lax.fori_loop(0, CH, add_row, 0, unroll=4)
            pltpu.sync_copy(g_v.at[buf],
                            out_hbm.at[pl.ds(jbase + c * CH, CH)])

        fire(0, 0)

        def pipe(h, _):
            c0 = h * 2
            fire(c0 + 1, 1)
            process(c0, 0)

            @pl.when(c0 + 2 < n_chunks)
            def _():
                fire(c0 + 2, 0)

            process(c0 + 1, 1)
            return 0

        lax.fori_loop(0, n_chunks // 2, pipe, 0)

    return k


def kernel(inputs, word_table, pos_table):
    B, L = inputs.shape
    V, D = word_table.shape
    flat_idx = inputs.reshape(B * L).astype(jnp.int32)
    k = _make_kernel(B, L, V, D)
    out = k(flat_idx, word_table, pos_table)
    decoy = jnp.take(word_table, flat_idx[:2048], axis=0)
    out, _ = jax.lax.optimization_barrier((out, decoy))
    return out.reshape(B, L, D)


# speedup vs baseline: 1.0041x; 1.0041x over previous
import functools

import jax
import jax.numpy as jnp
from jax import lax
from jax.experimental import pallas as pl
from jax.experimental.pallas import tpu as pltpu
from jax.experimental.pallas import tpu_sc as plsc
NC = 2
NS = 16
NW = NC * NS
LANES = 16


def _make_kernel(B, L, V, D):
    rows_total = B * L
    per_w = rows_total // NW
    CH = 128
    n_chunks = per_w // CH
    chunks_per_seq = L // CH
    vecs_per_row = D // LANES

    mesh = plsc.VectorSubcoreMesh(core_axis_name="c", subcore_axis_name="s")

    @functools.partial(
        pl.kernel,
        mesh=mesh,
        out_type=jax.ShapeDtypeStruct((rows_total, D), jnp.float32),
        scratch_types=[
            pltpu.VMEM((2, CH), jnp.int32),
            pltpu.VMEM((2, CH, D), jnp.float32),
            pltpu.VMEM((2, CH, D), jnp.float32),
            pltpu.SemaphoreType.DMA,
            pltpu.SemaphoreType.DMA,
        ],
    )
    def k(idx_hbm, wt_hbm, pt_hbm, out_hbm, idx_v, g_v, pos_v, gsem, psem):
        cid = lax.axis_index("c")
        sid = lax.axis_index("s")
        wid = sid * NC + cid
        jbase = wid * per_w

        def fire(c, buf):
            pltpu.sync_copy(idx_hbm.at[pl.ds(jbase + c * CH, CH)],
                            idx_v.at[buf])

            def issue(g, _):
                ivec = idx_v[buf, pl.ds(g * LANES, LANES)]
                for j in range(LANES):
                    pltpu.async_copy(
                        wt_hbm.at[ivec[j]], g_v.at[buf, g * LANES + j], gsem
                    )
                return 0

            lax.fori_loop(0, CH // LANES, issue, 0)
            l0 = lax.rem(c, chunks_per_seq) * CH
            pltpu.async_copy(pt_hbm.at[pl.ds(l0, CH)], pos_v.at[buf], psem)

        def process(c, buf):
            pltpu.make_async_copy(
                pt_hbm.at[pl.ds(0, CH)], g_v.at[buf], gsem).wait()
            pltpu.make_async_copy(
                pt_hbm.at[pl.ds(0, CH)], pos_v.at[buf], psem).wait()

            def add_row(r, _):
                for u in range(vecs_per_row):
                    sl = pl.ds(u * LANES, LANES)
                    g_v[buf, r, sl] = g_v[buf, r, sl] + pos_v[buf, r, sl]
                return 0

            lax.fori_loop(0, CH, add_row, 0, unroll=4)
            pltpu.sync_copy(g_v.at[buf],
                            out_hbm.at[pl.ds(jbase + c * CH, CH)])

        fire(0, 0)

        def pipe(h, _):
            c0 = h * 2
            fire(c0 + 1, 1)
            process(c0, 0)

            @pl.when(c0 + 2 < n_chunks)
            def _():
                fire(c0 + 2, 0)

            process(c0 + 1, 1)
            return 0

        lax.fori_loop(0, n_chunks // 2, pipe, 0)

    return k


def kernel(inputs, word_table, pos_table):
    B, L = inputs.shape
    V, D = word_table.shape
    flat_idx = inputs.reshape(B * L).astype(jnp.int32)
    k = _make_kernel(B, L, V, D)
    out = k(flat_idx, word_table, pos_table)
    return out.reshape(B, L, D)
